# Initial kernel scaffold; baseline (speedup 1.0000x reference)
import jax
import jax.numpy as jnp
from jax.experimental import pallas as pl


def kernel(agent_indices, z_local, patch_embed, W_near1, b_near1, W_near2, b_near2, W_inv1, b_inv1, W_inv2, b_inv2):
    raise NotImplementedError("write your pallas kernel here")


# speedup vs baseline: 4.6569x; 4.6569x over previous
import functools

import jax
import jax.numpy as jnp
from jax import lax
from jax.experimental import pallas as pl
from jax.experimental.pallas import tpu as pltpu
from jax.experimental.pallas import tpu_sc as plsc

_B = 16384
_N_PATCH = 9
_CODEBOOK = 4096
_PATCH_DIM = 32
_HIDDEN = 128

_NIDX = _B * _N_PATCH
_CHUNK = 128
_NROWS = _NIDX // _CHUNK
_NW = 32
_ROWS_PER_W = _NROWS // _NW
_FIRE = 9
_NQ = _ROWS_PER_W // _FIRE


def _sc_gather_body(table_hbm, idx_hbm, out_hbm, idx_v, rows_v, sem):
  wid = lax.axis_index("s") * 2 + lax.axis_index("c")
  base = wid * _ROWS_PER_W
  pltpu.sync_copy(idx_hbm.at[wid], idx_v)

  def quarter(q, carry):
    start = q * _FIRE
    copies = []
    for j in range(_FIRE):
      copies.append(
          pltpu.async_copy(
              table_hbm.at[idx_v.at[start + j]], rows_v.at[j], sem))
    for c in copies:
      c.wait()
    pltpu.sync_copy(rows_v, out_hbm.at[pl.ds(base + start, _FIRE)])
    return carry

  lax.fori_loop(0, _NQ, quarter, 0, unroll=False)


@functools.partial(jax.jit, static_argnames=())
def _sc_gather(table, idx2d):
  mesh = plsc.VectorSubcoreMesh(core_axis_name="c", subcore_axis_name="s")
  return pl.kernel(
      _sc_gather_body,
      out_type=jax.ShapeDtypeStruct((_NROWS, _CHUNK, _PATCH_DIM), jnp.float32),
      mesh=mesh,
      scratch_types=[
          pltpu.VMEM((_ROWS_PER_W, _CHUNK), jnp.int32),
          pltpu.VMEM((_FIRE, _CHUNK, _PATCH_DIM), jnp.float32),
          pltpu.SemaphoreType.DMA,
      ],
      compiler_params=pltpu.CompilerParams(use_tc_tiling_on_sc=False),
  )(table, idx2d)


def _mlp_body(x_ref, w1_ref, b1_ref, w2_ref, b2_ref, o_ref):
  h = jnp.dot(x_ref[...], w1_ref[...], preferred_element_type=jnp.float32)
  h = jnp.maximum(h + b1_ref[...], 0.0)
  o_ref[...] = (
      jnp.dot(h, w2_ref[...], preferred_element_type=jnp.float32)
      + b2_ref[...])


def _mlp_head(x, w1, b1, w2, b2, block_rows):
  n, k = x.shape
  m = w2.shape[1]
  grid = (n // block_rows,)
  return pl.pallas_call(
      _mlp_body,
      grid=grid,
      in_specs=[
          pl.BlockSpec((block_rows, k), lambda i: (i, 0)),
          pl.BlockSpec((k, _HIDDEN), lambda i: (0, 0)),
          pl.BlockSpec((1, _HIDDEN), lambda i: (0, 0)),
          pl.BlockSpec((_HIDDEN, m), lambda i: (0, 0)),
          pl.BlockSpec((1, m), lambda i: (0, 0)),
      ],
      out_specs=pl.BlockSpec((block_rows, m), lambda i: (i, 0)),
      out_shape=jax.ShapeDtypeStruct((n, m), jnp.float32),
  )(x, w1, b1, w2, b2)


def kernel(agent_indices, z_local, patch_embed, W_near1, b_near1, W_near2,
           b_near2, W_inv1, b_inv1, W_inv2, b_inv2):
  idx2d = agent_indices.reshape(_NW, _ROWS_PER_W, _CHUNK)
  gathered = _sc_gather(patch_embed, idx2d)
  patch_flat = gathered.reshape(_B, _N_PATCH * _PATCH_DIM)

  inv_logits = _mlp_head(
      z_local, W_inv1, b_inv1.reshape(1, -1), W_inv2, b_inv2.reshape(1, -1),
      block_rows=512)
  near_logits = _mlp_head(
      patch_flat, W_near1, b_near1.reshape(1, -1), W_near2,
      b_near2.reshape(1, -1), block_rows=2048)
  return (near_logits, inv_logits)


# speedup vs baseline: 4.9636x; 1.0659x over previous
import functools

import jax
import jax.numpy as jnp
from jax import lax
from jax.experimental import pallas as pl
from jax.experimental.pallas import tpu as pltpu
from jax.experimental.pallas import tpu_sc as plsc

_B = 16384
_N_PATCH = 9
_CODEBOOK = 4096
_PATCH_DIM = 32
_HIDDEN = 128

_NIDX = _B * _N_PATCH
_CHUNK = 128
_NROWS = _NIDX // _CHUNK
_NW = 32
_ROWS_PER_W = _NROWS // _NW
_FIRE = 9
_NQ = _ROWS_PER_W // _FIRE


def _sc_gather_body(table_hbm, idx_hbm, out_hbm, idx_v, rows_v, sem):
  wid = lax.axis_index("s") * 2 + lax.axis_index("c")
  base = wid * _ROWS_PER_W
  pltpu.sync_copy(idx_hbm.at[wid], idx_v)

  def quarter(q, carry):
    start = q * _FIRE
    copies = []
    for j in range(_FIRE):
      copies.append(
          pltpu.async_copy(
              table_hbm.at[idx_v.at[start + j]], rows_v.at[j], sem))
    for c in copies:
      c.wait()
    pltpu.sync_copy(rows_v, out_hbm.at[pl.ds(base + start, _FIRE)])
    return carry

  lax.fori_loop(0, _NQ, quarter, 0, unroll=False)


@functools.partial(jax.jit, static_argnames=())
def _sc_gather(table, idx2d):
  mesh = plsc.VectorSubcoreMesh(core_axis_name="c", subcore_axis_name="s")
  return pl.kernel(
      _sc_gather_body,
      out_type=jax.ShapeDtypeStruct((_NROWS, _CHUNK, _PATCH_DIM), jnp.float32),
      mesh=mesh,
      scratch_types=[
          pltpu.VMEM((_ROWS_PER_W, _CHUNK), jnp.int32),
          pltpu.VMEM((_FIRE, _CHUNK, _PATCH_DIM), jnp.float32),
          pltpu.SemaphoreType.DMA,
      ],
      compiler_params=pltpu.CompilerParams(use_tc_tiling_on_sc=False),
  )(table, idx2d)


def _mlp_body(x_ref, w1_ref, b1_ref, w2_ref, b2_ref, o_ref):
  h = jnp.dot(x_ref[...], w1_ref[...], preferred_element_type=jnp.float32)
  h = jnp.maximum(h + b1_ref[...], 0.0)
  o_ref[...] = (
      jnp.dot(h, w2_ref[...], preferred_element_type=jnp.float32)
      + b2_ref[...])


def _mlp_head(x, w1, b1, w2, b2, block_rows):
  n, k = x.shape
  m = w2.shape[1]
  grid = (n // block_rows,)
  return pl.pallas_call(
      _mlp_body,
      grid=grid,
      in_specs=[
          pl.BlockSpec((block_rows, k), lambda i: (i, 0)),
          pl.BlockSpec((k, _HIDDEN), lambda i: (0, 0)),
          pl.BlockSpec((1, _HIDDEN), lambda i: (0, 0)),
          pl.BlockSpec((_HIDDEN, m), lambda i: (0, 0)),
          pl.BlockSpec((1, m), lambda i: (0, 0)),
      ],
      out_specs=pl.BlockSpec((block_rows, m), lambda i: (i, 0)),
      out_shape=jax.ShapeDtypeStruct((n, m), jnp.float32),
  )(x, w1, b1, w2, b2)


def kernel(agent_indices, z_local, patch_embed, W_near1, b_near1, W_near2,
           b_near2, W_inv1, b_inv1, W_inv2, b_inv2):
  idx2d = agent_indices.reshape(_NW, _ROWS_PER_W, _CHUNK)
  gathered = _sc_gather(patch_embed, idx2d)
  patch_flat = gathered.reshape(_B, _N_PATCH * _PATCH_DIM)

  inv_logits = _mlp_head(
      z_local, W_inv1, b_inv1.reshape(1, -1), W_inv2, b_inv2.reshape(1, -1),
      block_rows=1024)
  near_logits = _mlp_head(
      patch_flat, W_near1, b_near1.reshape(1, -1), W_near2,
      b_near2.reshape(1, -1), block_rows=4096)
  return (near_logits, inv_logits)


# speedup vs baseline: 5.0204x; 1.0115x over previous
import functools

import jax
import jax.numpy as jnp
from jax import lax
from jax.experimental import pallas as pl
from jax.experimental.pallas import tpu as pltpu
from jax.experimental.pallas import tpu_sc as plsc

_B = 16384
_N_PATCH = 9
_CODEBOOK = 4096
_PATCH_DIM = 32
_HIDDEN = 128

_NIDX = _B * _N_PATCH
_CHUNK = 128
_NROWS = _NIDX // _CHUNK
_NW = 32
_ROWS_PER_W = _NROWS // _NW
_FIRE = 9
_NQ = _ROWS_PER_W // _FIRE


def _sc_gather_body(table_hbm, idx_hbm, out_hbm, idx_v, rows_v, sem):
  wid = lax.axis_index("s") * 2 + lax.axis_index("c")
  base = wid * _ROWS_PER_W
  pltpu.sync_copy(idx_hbm.at[wid], idx_v)

  def quarter(q, carry):
    start = q * _FIRE
    copies = []
    for j in range(_FIRE):
      copies.append(
          pltpu.async_copy(
              table_hbm.at[idx_v.at[start + j]], rows_v.at[j], sem))
    for c in copies:
      c.wait()
    pltpu.sync_copy(rows_v, out_hbm.at[pl.ds(base + start, _FIRE)])
    return carry

  lax.fori_loop(0, _NQ, quarter, 0, unroll=False)


@functools.partial(jax.jit, static_argnames=())
def _sc_gather(table, idx2d):
  mesh = plsc.VectorSubcoreMesh(core_axis_name="c", subcore_axis_name="s")
  return pl.kernel(
      _sc_gather_body,
      out_type=jax.ShapeDtypeStruct((_NROWS, _CHUNK, _PATCH_DIM), jnp.float32),
      mesh=mesh,
      scratch_types=[
          pltpu.VMEM((_ROWS_PER_W, _CHUNK), jnp.int32),
          pltpu.VMEM((_FIRE, _CHUNK, _PATCH_DIM), jnp.float32),
          pltpu.SemaphoreType.DMA,
      ],
      compiler_params=pltpu.CompilerParams(use_tc_tiling_on_sc=False),
  )(table, idx2d)


def _mlp_body(x_ref, w1_ref, b1_ref, w2_ref, b2_ref, o_ref):
  h = jnp.dot(x_ref[...], w1_ref[...], preferred_element_type=jnp.float32)
  h = jnp.maximum(h + b1_ref[...], 0.0)
  o_ref[...] = (
      jnp.dot(h, w2_ref[...], preferred_element_type=jnp.float32)
      + b2_ref[...])


def _mlp_head(x, w1, b1, w2, b2, block_rows):
  n, k = x.shape
  m = w2.shape[1]
  grid = (n // block_rows,)
  return pl.pallas_call(
      _mlp_body,
      grid=grid,
      in_specs=[
          pl.BlockSpec((block_rows, k), lambda i: (i, 0)),
          pl.BlockSpec((k, _HIDDEN), lambda i: (0, 0)),
          pl.BlockSpec((1, _HIDDEN), lambda i: (0, 0)),
          pl.BlockSpec((_HIDDEN, m), lambda i: (0, 0)),
          pl.BlockSpec((1, m), lambda i: (0, 0)),
      ],
      out_specs=pl.BlockSpec((block_rows, m), lambda i: (i, 0)),
      out_shape=jax.ShapeDtypeStruct((n, m), jnp.float32),
  )(x, w1, b1, w2, b2)


def kernel(agent_indices, z_local, patch_embed, W_near1, b_near1, W_near2,
           b_near2, W_inv1, b_inv1, W_inv2, b_inv2):
  idx2d = agent_indices.reshape(_NW, _ROWS_PER_W, _CHUNK)
  gathered = _sc_gather(patch_embed, idx2d)
  patch_flat = gathered.reshape(_B, _N_PATCH * _PATCH_DIM)

  inv_logits = _mlp_head(
      z_local, W_inv1, b_inv1.reshape(1, -1), W_inv2, b_inv2.reshape(1, -1),
      block_rows=2048)
  near_logits = _mlp_head(
      patch_flat, W_near1, b_near1.reshape(1, -1), W_near2,
      b_near2.reshape(1, -1), block_rows=8192)
  return (near_logits, inv_logits)


# speedup vs baseline: 5.1181x; 1.0195x over previous
import functools

import jax
import jax.numpy as jnp
from jax import lax
from jax.experimental import pallas as pl
from jax.experimental.pallas import tpu as pltpu
from jax.experimental.pallas import tpu_sc as plsc

_B = 16384
_PATCH_DIM = 32
_HIDDEN = 128
_NPLANE = 3
_CHUNK = 128
_NW = 32
_CPW = 16
_ROWS_PER_PLANE = 512


def _sc_gather_body(table_hbm, idx_hbm, out_hbm, idx_v, rows_v, sem):
  wid = lax.axis_index("s") * 2 + lax.axis_index("c")
  pltpu.sync_copy(idx_hbm.at[wid], idx_v)

  def plane(c, carry):
    copies = []
    for j in range(_CPW):
      copies.append(
          pltpu.async_copy(
              table_hbm.at[idx_v.at[c * _CPW + j]], rows_v.at[j], sem))
    for cp in copies:
      cp.wait()
    pltpu.sync_copy(rows_v, out_hbm.at[c, pl.ds(_CPW * wid, _CPW)])
    return carry

  lax.fori_loop(0, _NPLANE, plane, 0, unroll=False)


def _sc_gather(table, idx_wmaj):
  mesh = plsc.VectorSubcoreMesh(core_axis_name="c", subcore_axis_name="s")
  return pl.kernel(
      _sc_gather_body,
      out_type=jax.ShapeDtypeStruct(
          (_NPLANE, _ROWS_PER_PLANE, _CHUNK, _PATCH_DIM), jnp.float32),
      mesh=mesh,
      scratch_types=[
          pltpu.VMEM((_NPLANE * _CPW, _CHUNK), jnp.int32),
          pltpu.VMEM((_CPW, _CHUNK, _PATCH_DIM), jnp.float32),
          pltpu.SemaphoreType.DMA,
      ],
      compiler_params=pltpu.CompilerParams(use_tc_tiling_on_sc=False),
  )(table, idx_wmaj)


def _inv_body(x_ref, w1_ref, b1_ref, w2_ref, b2_ref, o_ref):
  h = jnp.dot(x_ref[...], w1_ref[...], preferred_element_type=jnp.float32)
  h = jnp.maximum(h + b1_ref[...], 0.0)
  o_ref[...] = (
      jnp.dot(h, w2_ref[...], preferred_element_type=jnp.float32)
      + b2_ref[...])


def _inv_head(x, w1, b1, w2, b2, block_rows):
  n, k = x.shape
  m = w2.shape[1]
  return pl.pallas_call(
      _inv_body,
      grid=(n // block_rows,),
      in_specs=[
          pl.BlockSpec((block_rows, k), lambda i: (i, 0)),
          pl.BlockSpec((k, _HIDDEN), lambda i: (0, 0)),
          pl.BlockSpec((1, _HIDDEN), lambda i: (0, 0)),
          pl.BlockSpec((_HIDDEN, m), lambda i: (0, 0)),
          pl.BlockSpec((1, m), lambda i: (0, 0)),
      ],
      out_specs=pl.BlockSpec((block_rows, m), lambda i: (i, 0)),
      out_shape=jax.ShapeDtypeStruct((n, m), jnp.float32),
  )(x, w1, b1, w2, b2)


def _near_body(x_ref, w1_ref, b1_ref, w2_ref, b2_ref, o_ref):
  h = jnp.dot(x_ref[0], w1_ref[0], preferred_element_type=jnp.float32)
  h += jnp.dot(x_ref[1], w1_ref[1], preferred_element_type=jnp.float32)
  h += jnp.dot(x_ref[2], w1_ref[2], preferred_element_type=jnp.float32)
  h = jnp.maximum(h + b1_ref[...], 0.0)
  o_ref[...] = (
      jnp.dot(h, w2_ref[...], preferred_element_type=jnp.float32)
      + b2_ref[...])


def _near_head(x, w1p, b1, w2, b2, block_rows):
  m = w2.shape[1]
  return pl.pallas_call(
      _near_body,
      grid=(_B // block_rows,),
      in_specs=[
          pl.BlockSpec((_NPLANE, block_rows, _HIDDEN), lambda i: (0, i, 0)),
          pl.BlockSpec((_NPLANE, _HIDDEN, _HIDDEN), lambda i: (0, 0, 0)),
          pl.BlockSpec((1, _HIDDEN), lambda i: (0, 0)),
          pl.BlockSpec((_HIDDEN, m), lambda i: (0, 0)),
          pl.BlockSpec((1, m), lambda i: (0, 0)),
      ],
      out_specs=pl.BlockSpec((block_rows, m), lambda i: (i, 0)),
      out_shape=jax.ShapeDtypeStruct((_B, m), jnp.float32),
  )(x, w1p, b1, w2, b2)


def kernel(agent_indices, z_local, patch_embed, W_near1, b_near1, W_near2,
           b_near2, W_inv1, b_inv1, W_inv2, b_inv2):
  a = agent_indices
  planes = jnp.stack(
      [a[:, 0:4], a[:, 4:8], jnp.broadcast_to(a[:, 8:9], (_B, 4))], axis=0)
  idx_wmaj = (
      planes.reshape(_NPLANE, _NW, _CPW, _CHUNK)
      .transpose(1, 0, 2, 3)
      .reshape(_NW, _NPLANE * _CPW, _CHUNK))

  gathered = _sc_gather(patch_embed, idx_wmaj)
  x = gathered.reshape(_NPLANE, _B, _HIDDEN)

  inv_logits = _inv_head(
      z_local, W_inv1, b_inv1.reshape(1, -1), W_inv2, b_inv2.reshape(1, -1),
      block_rows=2048)

  w1p = jnp.concatenate(
      [W_near1, jnp.zeros((_NPLANE * _HIDDEN - W_near1.shape[0], _HIDDEN),
                          jnp.float32)], axis=0).reshape(_NPLANE, _HIDDEN,
                                                         _HIDDEN)
  near_logits = _near_head(
      x, w1p, b_near1.reshape(1, -1), W_near2, b_near2.reshape(1, -1),
      block_rows=8192)
  return (near_logits, inv_logits)


# speedup vs baseline: 5.5919x; 1.0926x over previous
import functools

import jax
import jax.numpy as jnp
from jax import lax
from jax.experimental import pallas as pl
from jax.experimental.pallas import tpu as pltpu
from jax.experimental.pallas import tpu_sc as plsc

_B = 16384
_PATCH_DIM = 32
_HIDDEN = 128
_NPLANE = 3
_CHUNK = 128
_NW = 32
_CPW = 16
_ROWS_PER_PLANE = 512


def _sc_gather_body(table_hbm, idx_hbm, out_hbm, raw_v, idx_v, rows_v, sem):
  wid = lax.axis_index("s") * 2 + lax.axis_index("c")
  rows_per_w = _B // _NW
  pltpu.sync_copy(idx_hbm.at[pl.ds(rows_per_w * wid, rows_per_w)], raw_v)

  lane = lax.iota(jnp.int32, 16)
  lane_row = lax.shift_right_logical(lane, 2)
  lane_q = lax.bitwise_and(lane, 3)

  def build(j, carry):
    for v in range(8):
      rows = 32 * j + 4 * v + lane_row
      for c in range(_NPLANE):
        if c < 2:
          cols = 4 * c + lane_q
        else:
          cols = jnp.full((16,), 8, jnp.int32)
        vec = plsc.load_gather(raw_v, [rows, cols])
        idx_v[c, j, pl.ds(16 * v, 16)] = vec
    return carry

  lax.fori_loop(0, _CPW, build, 0, unroll=False)

  def plane(c, carry):
    copies = []
    for j in range(_CPW):
      copies.append(
          pltpu.async_copy(
              table_hbm.at[idx_v.at[c, j]], rows_v.at[j], sem))
    for cp in copies:
      cp.wait()
    pltpu.sync_copy(rows_v, out_hbm.at[c, pl.ds(_CPW * wid, _CPW)])
    return carry

  lax.fori_loop(0, _NPLANE, plane, 0, unroll=False)


def _sc_gather(table, idx_raw):
  mesh = plsc.VectorSubcoreMesh(core_axis_name="c", subcore_axis_name="s")
  return pl.kernel(
      _sc_gather_body,
      out_type=jax.ShapeDtypeStruct(
          (_NPLANE, _ROWS_PER_PLANE, _CHUNK, _PATCH_DIM), jnp.float32),
      mesh=mesh,
      scratch_types=[
          pltpu.VMEM((_B // _NW, 9), jnp.int32),
          pltpu.VMEM((_NPLANE, _CPW, _CHUNK), jnp.int32),
          pltpu.VMEM((_CPW, _CHUNK, _PATCH_DIM), jnp.float32),
          pltpu.SemaphoreType.DMA,
      ],
      compiler_params=pltpu.CompilerParams(
          use_tc_tiling_on_sc=False, needs_layout_passes=False),
  )(table, idx_raw)


def _inv_body(x_ref, w1_ref, b1_ref, w2_ref, b2_ref, o_ref):
  h = jnp.dot(x_ref[...], w1_ref[...], preferred_element_type=jnp.float32)
  h = jnp.maximum(h + b1_ref[...], 0.0)
  o_ref[...] = (
      jnp.dot(h, w2_ref[...], preferred_element_type=jnp.float32)
      + b2_ref[...])


def _inv_head(x, w1, b1, w2, b2, block_rows):
  n, k = x.shape
  m = w2.shape[1]
  return pl.pallas_call(
      _inv_body,
      grid=(n // block_rows,),
      in_specs=[
          pl.BlockSpec((block_rows, k), lambda i: (i, 0)),
          pl.BlockSpec((k, _HIDDEN), lambda i: (0, 0)),
          pl.BlockSpec((1, _HIDDEN), lambda i: (0, 0)),
          pl.BlockSpec((_HIDDEN, m), lambda i: (0, 0)),
          pl.BlockSpec((1, m), lambda i: (0, 0)),
      ],
      out_specs=pl.BlockSpec((block_rows, m), lambda i: (i, 0)),
      out_shape=jax.ShapeDtypeStruct((n, m), jnp.float32),
  )(x, w1, b1, w2, b2)


def _near_body(x_ref, w1_ref, b1_ref, w2_ref, b2_ref, o_ref):
  h = jnp.dot(x_ref[0], w1_ref[0], preferred_element_type=jnp.float32)
  h += jnp.dot(x_ref[1], w1_ref[1], preferred_element_type=jnp.float32)
  h += jnp.dot(x_ref[2], w1_ref[2], preferred_element_type=jnp.float32)
  h = jnp.maximum(h + b1_ref[...], 0.0)
  o_ref[...] = (
      jnp.dot(h, w2_ref[...], preferred_element_type=jnp.float32)
      + b2_ref[...])


def _near_head(x, w1p, b1, w2, b2, block_rows):
  m = w2.shape[1]
  return pl.pallas_call(
      _near_body,
      grid=(_B // block_rows,),
      in_specs=[
          pl.BlockSpec((_NPLANE, block_rows, _HIDDEN), lambda i: (0, i, 0)),
          pl.BlockSpec((_NPLANE, _HIDDEN, _HIDDEN), lambda i: (0, 0, 0)),
          pl.BlockSpec((1, _HIDDEN), lambda i: (0, 0)),
          pl.BlockSpec((_HIDDEN, m), lambda i: (0, 0)),
          pl.BlockSpec((1, m), lambda i: (0, 0)),
      ],
      out_specs=pl.BlockSpec((block_rows, m), lambda i: (i, 0)),
      out_shape=jax.ShapeDtypeStruct((_B, m), jnp.float32),
  )(x, w1p, b1, w2, b2)


def kernel(agent_indices, z_local, patch_embed, W_near1, b_near1, W_near2,
           b_near2, W_inv1, b_inv1, W_inv2, b_inv2):
  gathered = _sc_gather(patch_embed, agent_indices)
  x = gathered.reshape(_NPLANE, _B, _HIDDEN)

  inv_logits = _inv_head(
      z_local, W_inv1, b_inv1.reshape(1, -1), W_inv2, b_inv2.reshape(1, -1),
      block_rows=2048)

  w1p = jnp.concatenate(
      [W_near1, jnp.zeros((_NPLANE * _HIDDEN - W_near1.shape[0], _HIDDEN),
                          jnp.float32)], axis=0).reshape(_NPLANE, _HIDDEN,
                                                         _HIDDEN)
  near_logits = _near_head(
      x, w1p, b_near1.reshape(1, -1), W_near2, b_near2.reshape(1, -1),
      block_rows=8192)
  return (near_logits, inv_logits)


# speedup vs baseline: 6.5770x; 1.1762x over previous
import jax
import jax.numpy as jnp
from jax import lax
from jax.experimental import pallas as pl
from jax.experimental.pallas import tpu as pltpu
from jax.experimental.pallas import tpu_sc as plsc

_B = 16384
_PATCH_DIM = 32
_HIDDEN = 128
_NPLANE = 3
_CHUNK = 128
_NW = 32
_CPW = 16
_ROWS_PER_PLANE = 512


def _sc_gather_body(table_hbm, idx_hbm, out_hbm, raw_v, idx_v, rows_v, sem):
  wid = lax.axis_index("s") * 2 + lax.axis_index("c")
  rows_per_w = _B // _NW
  pltpu.sync_copy(idx_hbm.at[pl.ds(9 * rows_per_w * wid, 9 * rows_per_w)],
                  raw_v)

  lane = lax.iota(jnp.int32, 16)
  lane_row = lax.shift_right_logical(lane, 2)
  lane_q = lax.bitwise_and(lane, 3)

  def build(j, carry):
    for v in range(8):
      rows = 32 * j + 4 * v + lane_row
      for c in range(_NPLANE):
        if c < 2:
          cols = 4 * c + lane_q
        else:
          cols = jnp.full((16,), 8, jnp.int32)
        vec = plsc.load_gather(raw_v, [rows * 9 + cols])
        idx_v[c, j, pl.ds(16 * v, 16)] = vec
    return carry

  lax.fori_loop(0, _CPW, build, 0, unroll=False)

  def plane(c, carry):
    copies = []
    for j in range(_CPW):
      copies.append(
          pltpu.async_copy(
              table_hbm.at[idx_v.at[c, j]], rows_v.at[j], sem))
    for cp in copies:
      cp.wait()
    pltpu.sync_copy(rows_v, out_hbm.at[c, pl.ds(_CPW * wid, _CPW)])
    return carry

  lax.fori_loop(0, _NPLANE, plane, 0, unroll=False)


def _sc_gather(table, idx_flat):
  mesh = plsc.VectorSubcoreMesh(core_axis_name="c", subcore_axis_name="s")
  return pl.kernel(
      _sc_gather_body,
      out_type=jax.ShapeDtypeStruct(
          (_NPLANE, _ROWS_PER_PLANE, _CHUNK, _PATCH_DIM), jnp.float32),
      mesh=mesh,
      scratch_types=[
          pltpu.VMEM((9 * _B // _NW,), jnp.int32),
          pltpu.VMEM((_NPLANE, _CPW, _CHUNK), jnp.int32),
          pltpu.VMEM((_CPW, _CHUNK, _PATCH_DIM), jnp.float32),
          pltpu.SemaphoreType.DMA,
      ],
      compiler_params=pltpu.CompilerParams(
          use_tc_tiling_on_sc=False, needs_layout_passes=False),
  )(table, idx_flat)


def _inv_body(x_ref, w1_ref, b1_ref, w2t_ref, b2t_ref, o_ref):
  h = jnp.dot(x_ref[...], w1_ref[...], preferred_element_type=jnp.float32)
  h = jnp.maximum(h + b1_ref[...], 0.0)
  o_ref[...] = lax.dot_general(
      w2t_ref[...], h, (((1,), (1,)), ((), ())),
      preferred_element_type=jnp.float32) + b2t_ref[...]


def _inv_head(x, w1, b1, w2t, b2t, block_rows):
  n, k = x.shape
  m = w2t.shape[0]
  return pl.pallas_call(
      _inv_body,
      grid=(n // block_rows,),
      in_specs=[
          pl.BlockSpec((block_rows, k), lambda i: (i, 0)),
          pl.BlockSpec((k, _HIDDEN), lambda i: (0, 0)),
          pl.BlockSpec((1, _HIDDEN), lambda i: (0, 0)),
          pl.BlockSpec((m, _HIDDEN), lambda i: (0, 0)),
          pl.BlockSpec((m, 1), lambda i: (0, 0)),
      ],
      out_specs=pl.BlockSpec((m, block_rows), lambda i: (0, i)),
      out_shape=jax.ShapeDtypeStruct((m, n), jnp.float32),
  )(x, w1, b1, w2t, b2t)


def _near_body(x_ref, w1_ref, b1_ref, w2t_ref, b2t_ref, o_ref):
  h = jnp.dot(x_ref[0], w1_ref[0], preferred_element_type=jnp.float32)
  h += jnp.dot(x_ref[1], w1_ref[1], preferred_element_type=jnp.float32)
  h += jnp.dot(x_ref[2], w1_ref[2], preferred_element_type=jnp.float32)
  h = jnp.maximum(h + b1_ref[...], 0.0)
  o_ref[...] = lax.dot_general(
      w2t_ref[...], h, (((1,), (1,)), ((), ())),
      preferred_element_type=jnp.float32) + b2t_ref[...]


def _near_head(x, w1p, b1, w2t, b2t, block_rows):
  m = w2t.shape[0]
  return pl.pallas_call(
      _near_body,
      grid=(_B // block_rows,),
      in_specs=[
          pl.BlockSpec((_NPLANE, block_rows, _HIDDEN), lambda i: (0, i, 0)),
          pl.BlockSpec((_NPLANE, _HIDDEN, _HIDDEN), lambda i: (0, 0, 0)),
          pl.BlockSpec((1, _HIDDEN), lambda i: (0, 0)),
          pl.BlockSpec((m, _HIDDEN), lambda i: (0, 0)),
          pl.BlockSpec((m, 1), lambda i: (0, 0)),
      ],
      out_specs=pl.BlockSpec((m, block_rows), lambda i: (0, i)),
      out_shape=jax.ShapeDtypeStruct((m, _B), jnp.float32),
  )(x, w1p, b1, w2t, b2t)


def kernel(agent_indices, z_local, patch_embed, W_near1, b_near1, W_near2,
           b_near2, W_inv1, b_inv1, W_inv2, b_inv2):
  gathered = _sc_gather(patch_embed, agent_indices.reshape(-1))
  x = gathered.reshape(_NPLANE, _B, _HIDDEN)

  inv_t = _inv_head(
      z_local, W_inv1, b_inv1.reshape(1, -1), W_inv2.T,
      b_inv2.reshape(-1, 1), block_rows=2048)

  w1p = jnp.concatenate(
      [W_near1, jnp.zeros((_NPLANE * _HIDDEN - W_near1.shape[0], _HIDDEN),
                          jnp.float32)], axis=0).reshape(_NPLANE, _HIDDEN,
                                                         _HIDDEN)
  near_t = _near_head(
      x, w1p, b_near1.reshape(1, -1), W_near2.T, b_near2.reshape(-1, 1),
      block_rows=8192)
  return (near_t.T, inv_t.T)


# speedup vs baseline: 6.5853x; 1.0013x over previous
import jax
import jax.numpy as jnp
from jax import lax
from jax.experimental import pallas as pl
from jax.experimental.pallas import tpu as pltpu
from jax.experimental.pallas import tpu_sc as plsc

_B = 16384
_PATCH_DIM = 32
_HIDDEN = 128
_NPLANE = 3
_CHUNK = 128
_NW = 32
_CPW = 16
_ROWS_PER_PLANE = 512


def _sc_gather_body(table_hbm, idx_hbm, out_hbm, raw_v, idx_v, rows_v, sem):
  wid = lax.axis_index("s") * 2 + lax.axis_index("c")
  rows_per_w = _B // _NW
  pltpu.sync_copy(idx_hbm.at[pl.ds(9 * rows_per_w * wid, 9 * rows_per_w)],
                  raw_v)

  lane = lax.iota(jnp.int32, 16)
  lane_row = lax.shift_right_logical(lane, 2)
  lane_q = lax.bitwise_and(lane, 3)

  def build(j, carry):
    for v in range(8):
      rows = 32 * j + 4 * v + lane_row
      for c in range(_NPLANE):
        if c < 2:
          cols = 4 * c + lane_q
        else:
          cols = jnp.full((16,), 8, jnp.int32)
        vec = plsc.load_gather(raw_v, [rows * 9 + cols])
        idx_v[c, j, pl.ds(16 * v, 16)] = vec
    return carry

  lax.fori_loop(0, _CPW, build, 0, unroll=False)

  def plane(c, carry):
    copies = []
    for j in range(_CPW):
      copies.append(
          pltpu.async_copy(
              table_hbm.at[idx_v.at[c, j]], rows_v.at[j], sem))
    for cp in copies:
      cp.wait()
    pltpu.sync_copy(rows_v, out_hbm.at[c, pl.ds(_CPW * wid, _CPW)])
    return carry

  lax.fori_loop(0, _NPLANE, plane, 0, unroll=False)


def _sc_gather(table, idx_flat):
  mesh = plsc.VectorSubcoreMesh(core_axis_name="c", subcore_axis_name="s")
  return pl.kernel(
      _sc_gather_body,
      out_type=jax.ShapeDtypeStruct(
          (_NPLANE, _ROWS_PER_PLANE, _CHUNK, _PATCH_DIM), jnp.float32),
      mesh=mesh,
      scratch_types=[
          pltpu.VMEM((9 * _B // _NW,), jnp.int32),
          pltpu.VMEM((_NPLANE, _CPW, _CHUNK), jnp.int32),
          pltpu.VMEM((_CPW, _CHUNK, _PATCH_DIM), jnp.float32),
          pltpu.SemaphoreType.DMA,
      ],
      compiler_params=pltpu.CompilerParams(
          use_tc_tiling_on_sc=False, needs_layout_passes=False),
  )(table, idx_flat)


def _inv_body(x_ref, w1_ref, b1_ref, w2t_ref, b2t_ref, o_ref):
  h = jnp.dot(x_ref[...], w1_ref[...], preferred_element_type=jnp.float32)
  h = jnp.maximum(h + b1_ref[...], 0.0)
  o_ref[...] = lax.dot_general(
      w2t_ref[...], h, (((1,), (1,)), ((), ())),
      preferred_element_type=jnp.float32) + b2t_ref[...]


def _inv_head(x, w1, b1, w2t, b2t, block_rows):
  n, k = x.shape
  m = w2t.shape[0]
  return pl.pallas_call(
      _inv_body,
      grid=(n // block_rows,),
      in_specs=[
          pl.BlockSpec((block_rows, k), lambda i: (i, 0)),
          pl.BlockSpec((k, _HIDDEN), lambda i: (0, 0)),
          pl.BlockSpec((1, _HIDDEN), lambda i: (0, 0)),
          pl.BlockSpec((m, _HIDDEN), lambda i: (0, 0)),
          pl.BlockSpec((m, 1), lambda i: (0, 0)),
      ],
      out_specs=pl.BlockSpec((m, block_rows), lambda i: (0, i)),
      out_shape=jax.ShapeDtypeStruct((m, n), jnp.float32),
  )(x, w1, b1, w2t, b2t)


def _near_body(x_ref, w1_ref, b1_ref, w2t_ref, b2t_ref, o_ref):
  h = jnp.dot(x_ref[0], w1_ref[0], preferred_element_type=jnp.float32)
  h += jnp.dot(x_ref[1], w1_ref[1], preferred_element_type=jnp.float32)
  h += jnp.dot(x_ref[2], w1_ref[2], preferred_element_type=jnp.float32)
  h = jnp.maximum(h + b1_ref[...], 0.0)
  o_ref[...] = lax.dot_general(
      w2t_ref[...], h, (((1,), (1,)), ((), ())),
      preferred_element_type=jnp.float32) + b2t_ref[...]


def _near_head(x, w1p, b1, w2t, b2t, block_rows):
  m = w2t.shape[0]
  return pl.pallas_call(
      _near_body,
      grid=(_B // block_rows,),
      in_specs=[
          pl.BlockSpec((_NPLANE, block_rows, _HIDDEN), lambda i: (0, i, 0)),
          pl.BlockSpec((_NPLANE, _HIDDEN, _HIDDEN), lambda i: (0, 0, 0)),
          pl.BlockSpec((1, _HIDDEN), lambda i: (0, 0)),
          pl.BlockSpec((m, _HIDDEN), lambda i: (0, 0)),
          pl.BlockSpec((m, 1), lambda i: (0, 0)),
      ],
      out_specs=pl.BlockSpec((m, block_rows), lambda i: (0, i)),
      out_shape=jax.ShapeDtypeStruct((m, _B), jnp.float32),
  )(x, w1p, b1, w2t, b2t)


def kernel(agent_indices, z_local, patch_embed, W_near1, b_near1, W_near2,
           b_near2, W_inv1, b_inv1, W_inv2, b_inv2):
  idx_flat = lax.optimization_barrier(agent_indices.reshape(-1))
  gathered = _sc_gather(patch_embed, idx_flat)
  x = gathered.reshape(_NPLANE, _B, _HIDDEN)

  inv_t = _inv_head(
      z_local, W_inv1, b_inv1.reshape(1, -1), W_inv2.T,
      b_inv2.reshape(-1, 1), block_rows=2048)

  w1p = jnp.concatenate(
      [W_near1, jnp.zeros((_NPLANE * _HIDDEN - W_near1.shape[0], _HIDDEN),
                          jnp.float32)], axis=0).reshape(_NPLANE, _HIDDEN,
                                                         _HIDDEN)
  near_t = _near_head(
      x, w1p, b_near1.reshape(1, -1), W_near2.T, b_near2.reshape(-1, 1),
      block_rows=8192)
  return (near_t.T, inv_t.T)


# speedup vs baseline: 7.2631x; 1.1029x over previous
import jax
import jax.numpy as jnp
from jax import lax
from jax.experimental import pallas as pl
from jax.experimental.pallas import tpu as pltpu
from jax.experimental.pallas import tpu_sc as plsc

_B = 16384
_PATCH_DIM = 32
_HIDDEN = 128
_NPLANE = 3
_CHUNK = 128
_NW = 32
_CPW = 16
_ROWS_PER_PLANE = 512


def _sc_gather_body(table_hbm, idx_hbm, out_hbm, raw_v, idx_v, rows_v, sem):
  wid = lax.axis_index("s") * 2 + lax.axis_index("c")
  rows_per_w = _B // _NW
  pltpu.sync_copy(idx_hbm.at[:, pl.ds(rows_per_w * wid, rows_per_w)], raw_v)

  lane = lax.iota(jnp.int32, 16)
  lane_row = lax.shift_right_logical(lane, 2)
  lane_q = lax.bitwise_and(lane, 3)

  def build(j, carry):
    for v in range(8):
      rows = 32 * j + 4 * v + lane_row
      for c in range(_NPLANE):
        if c < 2:
          cols = 4 * c + lane_q
        else:
          cols = jnp.full((16,), 8, jnp.int32)
        vec = plsc.load_gather(raw_v, [cols, rows])
        idx_v[c, j, pl.ds(16 * v, 16)] = vec
    return carry

  lax.fori_loop(0, _CPW, build, 0, unroll=False)

  def plane(c, carry):
    copies = []
    for j in range(_CPW):
      copies.append(
          pltpu.async_copy(
              table_hbm.at[idx_v.at[c, j]], rows_v.at[j], sem))
    for cp in copies:
      cp.wait()
    pltpu.sync_copy(rows_v, out_hbm.at[c, pl.ds(_CPW * wid, _CPW)])
    return carry

  lax.fori_loop(0, _NPLANE, plane, 0, unroll=False)


def _sc_gather(table, idx_flat):
  mesh = plsc.VectorSubcoreMesh(core_axis_name="c", subcore_axis_name="s")
  return pl.kernel(
      _sc_gather_body,
      out_type=jax.ShapeDtypeStruct(
          (_NPLANE, _ROWS_PER_PLANE, _CHUNK, _PATCH_DIM), jnp.float32),
      mesh=mesh,
      scratch_types=[
          pltpu.VMEM((9, _B // _NW), jnp.int32),
          pltpu.VMEM((_NPLANE, _CPW, _CHUNK), jnp.int32),
          pltpu.VMEM((_CPW, _CHUNK, _PATCH_DIM), jnp.float32),
          pltpu.SemaphoreType.DMA,
      ],
      compiler_params=pltpu.CompilerParams(
          use_tc_tiling_on_sc=False, needs_layout_passes=False),
  )(table, idx_flat)


def _inv_body(x_ref, w1_ref, b1_ref, w2t_ref, b2t_ref, o_ref):
  h = jnp.dot(x_ref[...], w1_ref[...], preferred_element_type=jnp.float32)
  h = jnp.maximum(h + b1_ref[...], 0.0)
  o_ref[...] = lax.dot_general(
      w2t_ref[...], h, (((1,), (1,)), ((), ())),
      preferred_element_type=jnp.float32) + b2t_ref[...]


def _inv_head(x, w1, b1, w2t, b2t, block_rows):
  n, k = x.shape
  m = w2t.shape[0]
  return pl.pallas_call(
      _inv_body,
      grid=(n // block_rows,),
      in_specs=[
          pl.BlockSpec((block_rows, k), lambda i: (i, 0)),
          pl.BlockSpec((k, _HIDDEN), lambda i: (0, 0)),
          pl.BlockSpec((1, _HIDDEN), lambda i: (0, 0)),
          pl.BlockSpec((m, _HIDDEN), lambda i: (0, 0)),
          pl.BlockSpec((m, 1), lambda i: (0, 0)),
      ],
      out_specs=pl.BlockSpec((m, block_rows), lambda i: (0, i)),
      out_shape=jax.ShapeDtypeStruct((m, n), jnp.float32),
  )(x, w1, b1, w2t, b2t)


def _near_body(x_ref, w1_ref, b1_ref, w2t_ref, b2t_ref, o_ref):
  h = jnp.dot(x_ref[0], w1_ref[0], preferred_element_type=jnp.float32)
  h += jnp.dot(x_ref[1], w1_ref[1], preferred_element_type=jnp.float32)
  h += jnp.dot(x_ref[2], w1_ref[2], preferred_element_type=jnp.float32)
  h = jnp.maximum(h + b1_ref[...], 0.0)
  o_ref[...] = lax.dot_general(
      w2t_ref[...], h, (((1,), (1,)), ((), ())),
      preferred_element_type=jnp.float32) + b2t_ref[...]


def _near_head(x, w1p, b1, w2t, b2t, block_rows):
  m = w2t.shape[0]
  return pl.pallas_call(
      _near_body,
      grid=(_B // block_rows,),
      in_specs=[
          pl.BlockSpec((_NPLANE, block_rows, _HIDDEN), lambda i: (0, i, 0)),
          pl.BlockSpec((_NPLANE, _HIDDEN, _HIDDEN), lambda i: (0, 0, 0)),
          pl.BlockSpec((1, _HIDDEN), lambda i: (0, 0)),
          pl.BlockSpec((m, _HIDDEN), lambda i: (0, 0)),
          pl.BlockSpec((m, 1), lambda i: (0, 0)),
      ],
      out_specs=pl.BlockSpec((m, block_rows), lambda i: (0, i)),
      out_shape=jax.ShapeDtypeStruct((m, _B), jnp.float32),
  )(x, w1p, b1, w2t, b2t)


def kernel(agent_indices, z_local, patch_embed, W_near1, b_near1, W_near2,
           b_near2, W_inv1, b_inv1, W_inv2, b_inv2):
  gathered = _sc_gather(patch_embed, agent_indices.T)
  x = gathered.reshape(_NPLANE, _B, _HIDDEN)

  inv_t = _inv_head(
      z_local, W_inv1, b_inv1.reshape(1, -1), W_inv2.T,
      b_inv2.reshape(-1, 1), block_rows=2048)

  w1p = jnp.concatenate(
      [W_near1, jnp.zeros((_NPLANE * _HIDDEN - W_near1.shape[0], _HIDDEN),
                          jnp.float32)], axis=0).reshape(_NPLANE, _HIDDEN,
                                                         _HIDDEN)
  near_t = _near_head(
      x, w1p, b_near1.reshape(1, -1), W_near2.T, b_near2.reshape(-1, 1),
      block_rows=8192)
  return (near_t.T, inv_t.T)
